# baseline (speedup 1.0000x reference)
import functools

import jax
import jax.numpy as jnp
from jax import lax
from jax.experimental import pallas as pl
from jax.experimental.pallas import tpu as pltpu
from jax.experimental.pallas import tpu_sc as plsc

VOCAB = 100000
D = 128
B = 4096
L = 200
N = B * L
NC = 2
NS = 16
NW = NC * NS
PER_W = N // NW
BATCH_PER_W = B // NW
C = 128
NB = 4
CHUNKS = PER_W // C
GROUPS = CHUNKS // NB
PAD_ROW = VOCAB
NPAD = 8192


def _body(table_h, ids_h, len_h, out_h, ids_v, len_v, rows_v, *sems):
    gsems = sems[:NB]
    wsems = sems[NB:]
    wid = lax.axis_index("s") * NC + lax.axis_index("c")
    base = wid * PER_W

    pltpu.sync_copy(ids_h.at[pl.ds(base, PER_W)], ids_v)
    pltpu.sync_copy(len_h.at[pl.ds(wid * BATCH_PER_W, BATCH_PER_W)], len_v)

    def mask_body(i, carry):
        lane = lax.iota(jnp.int32, 16)
        off = i * 16
        pos = off + lane
        b_local = lax.shift_right_logical(pos * 10486, 21)
        l = pos - b_local * L
        b0 = jnp.minimum(lax.shift_right_logical(off * 10486, 21), BATCH_PER_W - 16)
        lens16 = len_v[pl.ds(b0, 16)]
        lens = lax.gather(
            lens16,
            (b_local - b0)[:, None],
            lax.GatherDimensionNumbers(
                offset_dims=(), collapsed_slice_dims=(0,), start_index_map=(0,)
            ),
            (1,),
            mode=lax.GatherScatterMode.PROMISE_IN_BOUNDS,
        )
        pad_idx = PAD_ROW + jnp.bitwise_and(pos + base, NPAD - 1)
        m = jnp.maximum(0, jnp.minimum(1, lens - l))
        raw = ids_v[pl.ds(off, 16)]
        ids_v[pl.ds(off, 16)] = pad_idx + (raw - pad_idx) * m
        return carry

    lax.fori_loop(0, PER_W // 16, mask_body, 0)

    def gather(b, g):
        return pltpu.make_async_copy(
            table_h.at[ids_v.at[pl.ds(g * C, C)]],
            rows_v.at[pl.ds(b * C, C)],
            gsems[b],
        )

    def write(b, g):
        return pltpu.make_async_copy(
            rows_v.at[pl.ds(b * C, C)],
            out_h.at[pl.ds(base + g * C, C)],
            wsems[b],
        )

    for b in range(NB):
        gather(b, b).start()

    def group_body(gi, carry):
        g0 = gi * NB
        for b in range(NB):
            gather(b, g0 + b).wait()
            write(b, g0 + b).start()
        for b in range(NB):
            write(b, g0 + b).wait()
            gather(b, g0 + NB + b).start()
        return carry

    lax.fori_loop(0, GROUPS - 1, group_body, 0)

    g0 = (GROUPS - 1) * NB
    for b in range(NB):
        gather(b, g0 + b).wait()
        write(b, g0 + b).start()
    for b in range(NB):
        write(b, g0 + b).wait()


_gather = functools.partial(
    pl.kernel,
    out_type=jax.ShapeDtypeStruct((N, D), jnp.float32),
    mesh=plsc.VectorSubcoreMesh(core_axis_name="c", subcore_axis_name="s"),
    scratch_types=(
        [
            pltpu.VMEM((PER_W,), jnp.int32),
            pltpu.VMEM((BATCH_PER_W,), jnp.int32),
            pltpu.VMEM((NB * C, D), jnp.float32),
        ]
        + [pltpu.SemaphoreType.DMA] * (2 * NB)
    ),
)(_body)


@jax.jit
def kernel(token_ids, ori_token_length, table):
    table_p = jnp.concatenate(
        [table, jnp.zeros((NPAD, D), jnp.float32)], axis=0
    )
    ids = token_ids.reshape(N).astype(jnp.int32)
    lens = ori_token_length.astype(jnp.int32)
    out = _gather(table_p, ids, lens)
    return out.reshape(B, L, D)


# speedup vs baseline: 1.0179x; 1.0179x over previous
import functools

import jax
import jax.numpy as jnp
from jax import lax
from jax.experimental import pallas as pl
from jax.experimental.pallas import tpu as pltpu
from jax.experimental.pallas import tpu_sc as plsc

VOCAB = 100000
D = 128
B = 4096
L = 200
N = B * L
NC = 2
NS = 16
NW = NC * NS
PER_W = N // NW
BATCH_PER_W = B // NW
C = 128
NB = 6
CHUNKS = PER_W // C
GROUPS = CHUNKS // NB
PAD_ROW = VOCAB
NPAD = 8192


def _body(table_h, ids_h, len_h, out_h, ids_v, len_v, rows_v, *sems):
    gsems = sems[:NB]
    wsems = sems[NB:]
    wid = lax.axis_index("s") * NC + lax.axis_index("c")
    base = wid * PER_W

    pltpu.sync_copy(ids_h.at[pl.ds(base, PER_W)], ids_v)
    pltpu.sync_copy(len_h.at[pl.ds(wid * BATCH_PER_W, BATCH_PER_W)], len_v)

    def mask_body(i, carry):
        lane = lax.iota(jnp.int32, 16)
        off = i * 16
        pos = off + lane
        b_local = lax.shift_right_logical(pos * 10486, 21)
        l = pos - b_local * L
        b0 = jnp.minimum(lax.shift_right_logical(off * 10486, 21), BATCH_PER_W - 16)
        lens16 = len_v[pl.ds(b0, 16)]
        lens = lax.gather(
            lens16,
            (b_local - b0)[:, None],
            lax.GatherDimensionNumbers(
                offset_dims=(), collapsed_slice_dims=(0,), start_index_map=(0,)
            ),
            (1,),
            mode=lax.GatherScatterMode.PROMISE_IN_BOUNDS,
        )
        pad_idx = PAD_ROW + jnp.bitwise_and(pos + base, NPAD - 1)
        m = jnp.maximum(0, jnp.minimum(1, lens - l))
        raw = ids_v[pl.ds(off, 16)]
        ids_v[pl.ds(off, 16)] = pad_idx + (raw - pad_idx) * m
        return carry

    lax.fori_loop(0, PER_W // 16, mask_body, 0)

    def gather(b, g):
        return pltpu.make_async_copy(
            table_h.at[ids_v.at[pl.ds(g * C, C)]],
            rows_v.at[pl.ds(b * C, C)],
            gsems[b],
        )

    def write(b, g):
        return pltpu.make_async_copy(
            rows_v.at[pl.ds(b * C, C)],
            out_h.at[pl.ds(base + g * C, C)],
            wsems[b],
        )

    for b in range(NB):
        gather(b, b).start()

    def group_body(gi, carry):
        g0 = gi * NB
        for b in range(NB):
            gather(b, g0 + b).wait()
            write(b, g0 + b).start()
        for b in range(NB):
            write(b, g0 + b).wait()
            gather(b, g0 + NB + b).start()
        return carry

    lax.fori_loop(0, GROUPS - 1, group_body, 0)

    g0 = (GROUPS - 1) * NB
    for b in range(NB):
        gather(b, g0 + b).wait()
        write(b, g0 + b).start()
    for b in range(NB):
        write(b, g0 + b).wait()


_gather = functools.partial(
    pl.kernel,
    out_type=jax.ShapeDtypeStruct((N, D), jnp.float32),
    mesh=plsc.VectorSubcoreMesh(core_axis_name="c", subcore_axis_name="s"),
    scratch_types=(
        [
            pltpu.VMEM((PER_W,), jnp.int32),
            pltpu.VMEM((BATCH_PER_W,), jnp.int32),
            pltpu.VMEM((NB * C, D), jnp.float32),
        ]
        + [pltpu.SemaphoreType.DMA] * (2 * NB)
    ),
)(_body)


@jax.jit
def kernel(token_ids, ori_token_length, table):
    table_p = jnp.concatenate(
        [table, jnp.zeros((NPAD, D), jnp.float32)], axis=0
    )
    ids = token_ids.reshape(N).astype(jnp.int32)
    lens = ori_token_length.astype(jnp.int32)
    out = _gather(table_p, ids, lens)
    return out.reshape(B, L, D)
